# baseline (speedup 1.0000x reference)
import functools

import numpy as np
import jax
import jax.numpy as jnp
from jax import lax
from jax.experimental import pallas as pl
from jax.experimental.pallas import tpu as pltpu
from jax.experimental.pallas import tpu_sc as plsc

_N = 10000
_E = 320000
_D = 128
_HD = 64
_HDP = 80
_L = 8
_ALPHA = 0.2
_THETA = 1.0

_NC = 2
_NS = 16
_EB = 128
_NB = 158
_ETILE = _NB * _EB
_EPAD = _NS * _ETILE
_GARBAGE = 10008
_ROWS_PAD = 10016
_NPD = 10240
_DW = 16
_SBE = 512
_DTILE = 20480
_NSBC = _DTILE // _NC // _SBE

_RB = 2000


def _sc_mesh():
    return plsc.VectorSubcoreMesh(
        core_axis_name="c", subcore_axis_name="s",
        num_cores=_NC, num_subcores=_NS)


def _deg_body(col_ref, deg_ref, colb, onesb, zbuf, obuf, deg_sh):
    cid = lax.axis_index("c")
    sid = lax.axis_index("s")
    pltpu.sync_copy(col_ref.at[sid, cid], colb)
    zeros16 = jnp.zeros((16,), jnp.float32)
    ones16 = jnp.full((16,), 1.0, jnp.float32)

    def _fill(r, carry):
        onesb[r, :] = ones16
        return carry
    lax.fori_loop(0, _SBE, _fill, 0)

    def _zfill(r, carry):
        zbuf[r, :] = zeros16
        return carry
    lax.fori_loop(0, _EB, _zfill, 0)

    zbase = sid * (_NPD // _NS)
    for i in range(_NPD // _NS // _EB):
        pltpu.sync_copy(zbuf, deg_sh.at[pl.ds(zbase + i * _EB, _EB)])
    plsc.subcore_barrier()

    def _scat(j, carry):
        pltpu.sync_copy(onesb, deg_sh.at[colb.at[j]], add=True)
        return carry
    lax.fori_loop(0, _NSBC, _scat, 0)

    plsc.subcore_barrier()
    pltpu.sync_copy(deg_sh.at[pl.ds(zbase, _NPD // _NS)], obuf)
    pltpu.sync_copy(obuf, deg_ref.at[cid, pl.ds(zbase, _NPD // _NS)])


@functools.cache
def _deg_kernel():
    return pl.kernel(
        _deg_body,
        out_type=jax.ShapeDtypeStruct((_NC, _NPD, _DW), jnp.float32),
        mesh=_sc_mesh(),
        compiler_params=pltpu.CompilerParams(use_tc_tiling_on_sc=False),
        scratch_types=[
            pltpu.VMEM((_NSBC, _SBE), jnp.int32),
            pltpu.VMEM((_SBE, _DW), jnp.float32),
            pltpu.VMEM((_EB, _DW), jnp.float32),
            pltpu.VMEM((_NPD // _NS, _DW), jnp.float32),
            pltpu.VMEM_SHARED((_NPD, _DW), jnp.float32),
        ],
    )


def _agg_body(hs_ref, row_ref, col_ref, agg_ref,
              rowb, colb, gbufa, gbufb, agg_sh, gsema, gsemb):
    cid = lax.axis_index("c")
    sid = lax.axis_index("s")

    zeros16 = jnp.zeros((16,), jnp.float32)

    def _zrow(r, carry):
        for k in range(_HDP // 16):
            gbufa[r, pl.ds(k * 16, 16)] = zeros16
        return carry
    lax.fori_loop(0, _EB, _zrow, 0)

    zbase = sid * (_ROWS_PAD // _NS)
    nfull = (_ROWS_PAD // _NS) // _EB
    rem = (_ROWS_PAD // _NS) - nfull * _EB
    for i in range(nfull):
        pltpu.sync_copy(gbufa, agg_sh.at[pl.ds(zbase + i * _EB, _EB)])
    pltpu.sync_copy(gbufa.at[pl.ds(0, rem)],
                    agg_sh.at[pl.ds(zbase + nfull * _EB, rem)])
    plsc.subcore_barrier()

    def _run(table):
        pltpu.sync_copy(row_ref.at[sid], rowb)
        pltpu.sync_copy(col_ref.at[sid], colb)

        pltpu.async_copy(table.at[rowb.at[0]], gbufa, gsema)

        def _body(jj, carry):
            j = jj * 2
            pltpu.async_copy(table.at[rowb.at[j + 1]], gbufb, gsemb)
            pltpu.make_async_copy(
                table.at[rowb.at[0]], gbufa, gsema).wait()
            pltpu.sync_copy(gbufa, agg_sh.at[colb.at[j]], add=True)

            @pl.when(jj < _NB // 2 - 1)
            def _():
                pltpu.async_copy(table.at[rowb.at[j + 2]], gbufa, gsema)

            pltpu.make_async_copy(
                table.at[rowb.at[0]], gbufb, gsemb).wait()
            pltpu.sync_copy(gbufb, agg_sh.at[colb.at[j + 1]], add=True)
            return carry
        lax.fori_loop(0, _NB // 2, _body, 0)

    @pl.when(cid == 0)
    def _():
        _run(hs_ref.at[0])

    @pl.when(cid == 1)
    def _():
        _run(hs_ref.at[1])

    plsc.subcore_barrier()
    rbase = sid * (_N // _NS)
    pltpu.sync_copy(agg_sh.at[pl.ds(rbase, _N // _NS)],
                    agg_ref.at[cid, pl.ds(rbase, _N // _NS)])


@functools.cache
def _agg_kernel():
    return pl.kernel(
        _agg_body,
        out_type=jax.ShapeDtypeStruct((_NC, _N, _HDP), jnp.float32),
        mesh=_sc_mesh(),
        compiler_params=pltpu.CompilerParams(use_tc_tiling_on_sc=False),
        scratch_types=[
            pltpu.VMEM((_NB, _EB), jnp.int32),
            pltpu.VMEM((_NB, _EB), jnp.int32),
            pltpu.VMEM((_EB, _HDP), jnp.float32),
            pltpu.VMEM((_EB, _HDP), jnp.float32),
            pltpu.VMEM_SHARED((_ROWS_PAD, _HDP), jnp.float32),
            pltpu.SemaphoreType.DMA,
            pltpu.SemaphoreType.DMA,
        ],
    )


def _first_body(x_ref, w_ref, b_ref, deg_ref, h0_ref, hs_ref, dinv_ref):
    h = lax.dot_general(x_ref[...], w_ref[...], (((1,), (1,)), ((), ())),
                        preferred_element_type=jnp.float32)
    h = jnp.maximum(h + b_ref[...], 0.0)
    dinv = lax.rsqrt(deg_ref[0] + deg_ref[1] + 1.0)
    h0_ref[...] = h
    hs = h * dinv
    zpad = jnp.zeros((hs.shape[0], _HDP - _HD), jnp.float32)
    hs_ref[0] = jnp.concatenate([hs[:, :_HD], zpad], axis=1)
    hs_ref[1] = jnp.concatenate([hs[:, _HD:], zpad], axis=1)
    dinv_ref[...] = dinv


@functools.cache
def _first_kernel():
    return pl.pallas_call(
        _first_body,
        grid=(_N // _RB,),
        in_specs=[
            pl.BlockSpec((_RB, _D), lambda i: (i, 0)),
            pl.BlockSpec((_D, _D), lambda i: (0, 0)),
            pl.BlockSpec((1, _D), lambda i: (0, 0)),
            pl.BlockSpec((_NC, _RB, 1), lambda i: (0, i, 0)),
        ],
        out_specs=[
            pl.BlockSpec((_RB, _D), lambda i: (i, 0)),
            pl.BlockSpec((_NC, _RB, _HDP), lambda i: (0, i, 0)),
            pl.BlockSpec((_RB, 1), lambda i: (i, 0)),
        ],
        out_shape=[
            jax.ShapeDtypeStruct((_N, _D), jnp.float32),
            jax.ShapeDtypeStruct((_NC, _N, _HDP), jnp.float32),
            jax.ShapeDtypeStruct((_N, 1), jnp.float32),
        ],
    )


def _layer_body(agg_ref, hsp_ref, x0_ref, dinv_ref, w_ref, g_ref, be_ref,
                out_ref, *, bcoef, last):
    a = jnp.concatenate(
        [agg_ref[0, :, :_HD] + hsp_ref[0, :, :_HD],
         agg_ref[1, :, :_HD] + hsp_ref[1, :, :_HD]], axis=1)
    dinv = dinv_ref[...]
    support = (1.0 - _ALPHA) * (a * dinv) + _ALPHA * x0_ref[...]
    out = (1.0 - bcoef) * support + bcoef * lax.dot_general(
        support, w_ref[...], (((1,), (0,)), ((), ())),
        preferred_element_type=jnp.float32)
    mu = jnp.mean(out, axis=-1, keepdims=True)
    d = out - mu
    var = jnp.mean(d * d, axis=-1, keepdims=True)
    out = d * lax.rsqrt(var + 1e-5) * g_ref[...] + be_ref[...]
    if last:
        out_ref[...] = out
    else:
        out = jnp.maximum(out, 0.0)
        hs = out * dinv
        zpad = jnp.zeros((hs.shape[0], _HDP - _HD), jnp.float32)
        out_ref[0] = jnp.concatenate([hs[:, :_HD], zpad], axis=1)
        out_ref[1] = jnp.concatenate([hs[:, _HD:], zpad], axis=1)


@functools.cache
def _layer_kernel(l):
    bcoef = float(np.log(_THETA / (l + 1) + 1.0))
    last = l == _L - 1
    if last:
        out_spec = pl.BlockSpec((_RB, _D), lambda i: (i, 0))
        out_shape = jax.ShapeDtypeStruct((_N, _D), jnp.float32)
    else:
        out_spec = pl.BlockSpec((_NC, _RB, _HDP), lambda i: (0, i, 0))
        out_shape = jax.ShapeDtypeStruct((_NC, _N, _HDP), jnp.float32)
    return pl.pallas_call(
        functools.partial(_layer_body, bcoef=bcoef, last=last),
        grid=(_N // _RB,),
        in_specs=[
            pl.BlockSpec((_NC, _RB, _HDP), lambda i: (0, i, 0)),
            pl.BlockSpec((_NC, _RB, _HDP), lambda i: (0, i, 0)),
            pl.BlockSpec((_RB, _D), lambda i: (i, 0)),
            pl.BlockSpec((_RB, 1), lambda i: (i, 0)),
            pl.BlockSpec((_D, _D), lambda i: (0, 0)),
            pl.BlockSpec((1, _D), lambda i: (0, 0)),
            pl.BlockSpec((1, _D), lambda i: (0, 0)),
        ],
        out_specs=out_spec,
        out_shape=out_shape,
    )


def kernel(x, edge_index, W_in, b_in, conv_W, ln_gamma, ln_beta):
    per_w = _E // _NS
    rowp = jnp.pad(edge_index[0].reshape(_NS, per_w),
                   ((0, 0), (0, _ETILE - per_w))
                   ).reshape(_NS, _NB, _EB)
    colp = jnp.pad(edge_index[1].reshape(_NS, per_w),
                   ((0, 0), (0, _ETILE - per_w)),
                   constant_values=_GARBAGE
                   ).reshape(_NS, _NB, _EB)

    colp2 = jnp.pad(colp.reshape(_NS, _ETILE),
                    ((0, 0), (0, _DTILE - _ETILE)),
                    constant_values=_GARBAGE).reshape(_NS, _NC, _NSBC, _SBE)
    deg = _deg_kernel()(colp2)[:, :_N, 0:1]
    h0, hs, dinv = _first_kernel()(x, W_in, b_in.reshape(1, _D), deg)

    out = None
    for l in range(_L):
        agg = _agg_kernel()(hs, rowp, colp)
        step = _layer_kernel(l)(
            agg, hs, h0, dinv, conv_W[l],
            ln_gamma[l].reshape(1, _D), ln_beta[l].reshape(1, _D))
        if l == _L - 1:
            out = step
        else:
            hs = step
    return out


# speedup vs baseline: 1.1828x; 1.1828x over previous
import functools

import numpy as np
import jax
import jax.numpy as jnp
from jax import lax
from jax.experimental import pallas as pl
from jax.experimental.pallas import tpu as pltpu
from jax.experimental.pallas import tpu_sc as plsc

_N = 10000
_E = 320000
_D = 128
_HD = 64
_L = 8
_ALPHA = 0.2
_THETA = 1.0

_NC = 2
_NS = 16
_EB = 128
_NB = 158
_ETILE = _NB * _EB
_EPAD = _NS * _ETILE
_GARBAGE = 10008
_ROWS_PAD = 10016
_NPD = 10240
_DW = 16
_SBE = 512
_DTILE = 20480
_NSBC = _DTILE // _NC // _SBE

_RB = 2000


def _sc_mesh():
    return plsc.VectorSubcoreMesh(
        core_axis_name="c", subcore_axis_name="s",
        num_cores=_NC, num_subcores=_NS)


def _deg_body(col_ref, deg_ref, colb, onesb, zbuf, obuf, deg_sh):
    cid = lax.axis_index("c")
    sid = lax.axis_index("s")
    pltpu.sync_copy(col_ref.at[sid, cid], colb)
    zeros16 = jnp.zeros((16,), jnp.float32)
    ones16 = jnp.full((16,), 1.0, jnp.float32)

    def _fill(r, carry):
        onesb[r, :] = ones16
        return carry
    lax.fori_loop(0, _SBE, _fill, 0)

    def _zfill(r, carry):
        zbuf[r, :] = zeros16
        return carry
    lax.fori_loop(0, _EB, _zfill, 0)

    zbase = sid * (_NPD // _NS)
    for i in range(_NPD // _NS // _EB):
        pltpu.sync_copy(zbuf, deg_sh.at[pl.ds(zbase + i * _EB, _EB)])
    plsc.subcore_barrier()

    def _scat(j, carry):
        pltpu.sync_copy(onesb, deg_sh.at[colb.at[j]], add=True)
        return carry
    lax.fori_loop(0, _NSBC, _scat, 0)

    plsc.subcore_barrier()
    pltpu.sync_copy(deg_sh.at[pl.ds(zbase, _NPD // _NS)], obuf)
    pltpu.sync_copy(obuf, deg_ref.at[cid, pl.ds(zbase, _NPD // _NS)])


@functools.cache
def _deg_kernel():
    return pl.kernel(
        _deg_body,
        out_type=jax.ShapeDtypeStruct((_NC, _NPD, _DW), jnp.float32),
        mesh=_sc_mesh(),
        compiler_params=pltpu.CompilerParams(use_tc_tiling_on_sc=False),
        scratch_types=[
            pltpu.VMEM((_NSBC, _SBE), jnp.int32),
            pltpu.VMEM((_SBE, _DW), jnp.float32),
            pltpu.VMEM((_EB, _DW), jnp.float32),
            pltpu.VMEM((_NPD // _NS, _DW), jnp.float32),
            pltpu.VMEM_SHARED((_NPD, _DW), jnp.float32),
        ],
    )


def _agg_body(hs_ref, row_ref, col_ref, agg_ref,
              rowb, colb, gbufa, gbufb, agg_sh, gsema, gsemb):
    cid = lax.axis_index("c")
    sid = lax.axis_index("s")

    zeros16 = jnp.zeros((16,), jnp.float32)

    def _zrow(r, carry):
        for k in range(_HD // 16):
            gbufa[r, pl.ds(k * 16, 16)] = zeros16
        return carry
    lax.fori_loop(0, _EB, _zrow, 0)

    zbase = sid * (_ROWS_PAD // _NS)
    nfull = (_ROWS_PAD // _NS) // _EB
    rem = (_ROWS_PAD // _NS) - nfull * _EB
    for i in range(nfull):
        pltpu.sync_copy(gbufa, agg_sh.at[pl.ds(zbase + i * _EB, _EB)])
    pltpu.sync_copy(gbufa.at[pl.ds(0, rem)],
                    agg_sh.at[pl.ds(zbase + nfull * _EB, rem)])
    plsc.subcore_barrier()

    def _run(table):
        pltpu.sync_copy(row_ref.at[sid], rowb)
        pltpu.sync_copy(col_ref.at[sid], colb)

        pltpu.async_copy(table.at[rowb.at[0]], gbufa, gsema)

        def _body(jj, carry):
            j = jj * 2
            pltpu.async_copy(table.at[rowb.at[j + 1]], gbufb, gsemb)
            pltpu.make_async_copy(
                table.at[rowb.at[0]], gbufa, gsema).wait()
            pltpu.sync_copy(gbufa, agg_sh.at[colb.at[j]], add=True)

            @pl.when(jj < _NB // 2 - 1)
            def _():
                pltpu.async_copy(table.at[rowb.at[j + 2]], gbufa, gsema)

            pltpu.make_async_copy(
                table.at[rowb.at[0]], gbufb, gsemb).wait()
            pltpu.sync_copy(gbufb, agg_sh.at[colb.at[j + 1]], add=True)
            return carry
        lax.fori_loop(0, _NB // 2, _body, 0)

    @pl.when(cid == 0)
    def _():
        _run(hs_ref.at[0])

    @pl.when(cid == 1)
    def _():
        _run(hs_ref.at[1])

    plsc.subcore_barrier()
    rbase = sid * (_N // _NS)
    pltpu.sync_copy(agg_sh.at[pl.ds(rbase, _N // _NS)],
                    agg_ref.at[cid, pl.ds(rbase, _N // _NS)])


@functools.cache
def _agg_kernel():
    return pl.kernel(
        _agg_body,
        out_type=jax.ShapeDtypeStruct((_NC, _N, _HD), jnp.float32),
        mesh=_sc_mesh(),
        compiler_params=pltpu.CompilerParams(use_tc_tiling_on_sc=False),
        scratch_types=[
            pltpu.VMEM((_NB, _EB), jnp.int32),
            pltpu.VMEM((_NB, _EB), jnp.int32),
            pltpu.VMEM((_EB, _HD), jnp.float32),
            pltpu.VMEM((_EB, _HD), jnp.float32),
            pltpu.VMEM_SHARED((_ROWS_PAD, _HD), jnp.float32),
            pltpu.SemaphoreType.DMA,
            pltpu.SemaphoreType.DMA,
        ],
    )


def _first_body(x_ref, w_ref, b_ref, deg_ref, h0_ref, hs_ref, dinv_ref):
    h = lax.dot_general(x_ref[...], w_ref[...], (((1,), (1,)), ((), ())),
                        preferred_element_type=jnp.float32)
    h = jnp.maximum(h + b_ref[...], 0.0)
    dinv = lax.rsqrt(deg_ref[0] + deg_ref[1] + 1.0)
    h0_ref[...] = h
    hs = h * dinv
    hs_ref[0] = hs[:, :_HD]
    hs_ref[1] = hs[:, _HD:]
    dinv_ref[...] = dinv


@functools.cache
def _first_kernel():
    return pl.pallas_call(
        _first_body,
        grid=(_N // _RB,),
        in_specs=[
            pl.BlockSpec((_RB, _D), lambda i: (i, 0)),
            pl.BlockSpec((_D, _D), lambda i: (0, 0)),
            pl.BlockSpec((1, _D), lambda i: (0, 0)),
            pl.BlockSpec((_NC, _RB, 1), lambda i: (0, i, 0)),
        ],
        out_specs=[
            pl.BlockSpec((_RB, _D), lambda i: (i, 0)),
            pl.BlockSpec((_NC, _RB, _HD), lambda i: (0, i, 0)),
            pl.BlockSpec((_RB, 1), lambda i: (i, 0)),
        ],
        out_shape=[
            jax.ShapeDtypeStruct((_N, _D), jnp.float32),
            jax.ShapeDtypeStruct((_NC, _N, _HD), jnp.float32),
            jax.ShapeDtypeStruct((_N, 1), jnp.float32),
        ],
    )


def _layer_body(agg_ref, hsp_ref, x0_ref, dinv_ref, w_ref, g_ref, be_ref,
                out_ref, *, bcoef, last):
    a = jnp.concatenate(
        [agg_ref[0] + hsp_ref[0], agg_ref[1] + hsp_ref[1]], axis=1)
    dinv = dinv_ref[...]
    support = (1.0 - _ALPHA) * (a * dinv) + _ALPHA * x0_ref[...]
    out = (1.0 - bcoef) * support + bcoef * lax.dot_general(
        support, w_ref[...], (((1,), (0,)), ((), ())),
        preferred_element_type=jnp.float32)
    mu = jnp.mean(out, axis=-1, keepdims=True)
    d = out - mu
    var = jnp.mean(d * d, axis=-1, keepdims=True)
    out = d * lax.rsqrt(var + 1e-5) * g_ref[...] + be_ref[...]
    if last:
        out_ref[...] = out
    else:
        out = jnp.maximum(out, 0.0)
        hs = out * dinv
        out_ref[0] = hs[:, :_HD]
        out_ref[1] = hs[:, _HD:]


@functools.cache
def _layer_kernel(l):
    bcoef = float(np.log(_THETA / (l + 1) + 1.0))
    last = l == _L - 1
    if last:
        out_spec = pl.BlockSpec((_RB, _D), lambda i: (i, 0))
        out_shape = jax.ShapeDtypeStruct((_N, _D), jnp.float32)
    else:
        out_spec = pl.BlockSpec((_NC, _RB, _HD), lambda i: (0, i, 0))
        out_shape = jax.ShapeDtypeStruct((_NC, _N, _HD), jnp.float32)
    return pl.pallas_call(
        functools.partial(_layer_body, bcoef=bcoef, last=last),
        grid=(_N // _RB,),
        in_specs=[
            pl.BlockSpec((_NC, _RB, _HD), lambda i: (0, i, 0)),
            pl.BlockSpec((_NC, _RB, _HD), lambda i: (0, i, 0)),
            pl.BlockSpec((_RB, _D), lambda i: (i, 0)),
            pl.BlockSpec((_RB, 1), lambda i: (i, 0)),
            pl.BlockSpec((_D, _D), lambda i: (0, 0)),
            pl.BlockSpec((1, _D), lambda i: (0, 0)),
            pl.BlockSpec((1, _D), lambda i: (0, 0)),
        ],
        out_specs=out_spec,
        out_shape=out_shape,
    )


def kernel(x, edge_index, W_in, b_in, conv_W, ln_gamma, ln_beta):
    per_w = _E // _NS
    rowp = jnp.pad(edge_index[0].reshape(_NS, per_w),
                   ((0, 0), (0, _ETILE - per_w))
                   ).reshape(_NS, _NB, _EB)
    colp = jnp.pad(edge_index[1].reshape(_NS, per_w),
                   ((0, 0), (0, _ETILE - per_w)),
                   constant_values=_GARBAGE
                   ).reshape(_NS, _NB, _EB)

    colp2 = jnp.pad(colp.reshape(_NS, _ETILE),
                    ((0, 0), (0, _DTILE - _ETILE)),
                    constant_values=_GARBAGE).reshape(_NS, _NC, _NSBC, _SBE)
    deg = _deg_kernel()(colp2)[:, :_N, 0:1]
    h0, hs, dinv = _first_kernel()(x, W_in, b_in.reshape(1, _D), deg)

    out = None
    for l in range(_L):
        agg = _agg_kernel()(hs, rowp, colp)
        step = _layer_kernel(l)(
            agg, hs, h0, dinv, conv_W[l],
            ln_gamma[l].reshape(1, _D), ln_beta[l].reshape(1, _D))
        if l == _L - 1:
            out = step
        else:
            hs = step
    return out


# speedup vs baseline: 1.1954x; 1.0106x over previous
import functools

import numpy as np
import jax
import jax.numpy as jnp
from jax import lax
from jax.experimental import pallas as pl
from jax.experimental.pallas import tpu as pltpu
from jax.experimental.pallas import tpu_sc as plsc

_N = 10000
_E = 320000
_D = 128
_HD = 64
_L = 8
_ALPHA = 0.2
_THETA = 1.0

_NC = 2
_NS = 16
_EB = 128
_NB = 158
_ETILE = _NB * _EB
_EPAD = _NS * _ETILE
_GARBAGE = 10008
_ROWS_PAD = 10016
_NPD = 10240
_DW = 16
_SBE = 512
_DTILE = 20480
_NSBC = _DTILE // _NC // _SBE

_RB = 2000


def _sc_mesh():
    return plsc.VectorSubcoreMesh(
        core_axis_name="c", subcore_axis_name="s",
        num_cores=_NC, num_subcores=_NS)


def _deg_body(col_ref, deg_ref, colb, onesb, zbuf, obuf, deg_sh):
    cid = lax.axis_index("c")
    sid = lax.axis_index("s")
    pltpu.sync_copy(col_ref.at[sid, cid], colb)
    zeros16 = jnp.zeros((16,), jnp.float32)
    ones16 = jnp.full((16,), 1.0, jnp.float32)

    def _fill(r, carry):
        onesb[r, :] = ones16
        return carry
    lax.fori_loop(0, _SBE, _fill, 0)

    def _zfill(r, carry):
        zbuf[r, :] = zeros16
        return carry
    lax.fori_loop(0, _EB, _zfill, 0)

    zbase = sid * (_NPD // _NS)
    for i in range(_NPD // _NS // _EB):
        pltpu.sync_copy(zbuf, deg_sh.at[pl.ds(zbase + i * _EB, _EB)])
    plsc.subcore_barrier()

    def _scat(j, carry):
        pltpu.sync_copy(onesb, deg_sh.at[colb.at[j]], add=True)
        return carry
    lax.fori_loop(0, _NSBC, _scat, 0)

    plsc.subcore_barrier()
    pltpu.sync_copy(deg_sh.at[pl.ds(zbase, _NPD // _NS)], obuf)
    pltpu.sync_copy(obuf, deg_ref.at[cid, pl.ds(zbase, _NPD // _NS)])


@functools.cache
def _deg_kernel():
    return pl.kernel(
        _deg_body,
        out_type=jax.ShapeDtypeStruct((_NC, _NPD, _DW), jnp.float32),
        mesh=_sc_mesh(),
        compiler_params=pltpu.CompilerParams(use_tc_tiling_on_sc=False),
        scratch_types=[
            pltpu.VMEM((_NSBC, _SBE), jnp.int32),
            pltpu.VMEM((_SBE, _DW), jnp.float32),
            pltpu.VMEM((_EB, _DW), jnp.float32),
            pltpu.VMEM((_NPD // _NS, _DW), jnp.float32),
            pltpu.VMEM_SHARED((_NPD, _DW), jnp.float32),
        ],
    )


def _agg_body(hs_ref, row_ref, col_ref, agg_ref,
              rowb, colb, gbufa, gbufb, agg_sh, gsema, gsemb, isem):
    cid = lax.axis_index("c")
    sid = lax.axis_index("s")
    pltpu.async_copy(row_ref.at[sid], rowb, isem)
    pltpu.async_copy(col_ref.at[sid], colb, isem)

    zeros16 = jnp.zeros((16,), jnp.float32)

    def _zrow(r, carry):
        for k in range(_HD // 16):
            gbufa[r, pl.ds(k * 16, 16)] = zeros16
        return carry
    lax.fori_loop(0, _EB, _zrow, 0)

    zbase = sid * (_ROWS_PAD // _NS)
    nfull = (_ROWS_PAD // _NS) // _EB
    rem = (_ROWS_PAD // _NS) - nfull * _EB
    for i in range(nfull):
        pltpu.sync_copy(gbufa, agg_sh.at[pl.ds(zbase + i * _EB, _EB)])
    pltpu.sync_copy(gbufa.at[pl.ds(0, rem)],
                    agg_sh.at[pl.ds(zbase + nfull * _EB, rem)])
    pltpu.make_async_copy(row_ref.at[sid], rowb, isem).wait()
    pltpu.make_async_copy(col_ref.at[sid], colb, isem).wait()
    plsc.subcore_barrier()

    def _run(table):
        pltpu.async_copy(table.at[rowb.at[0]], gbufa, gsema)

        def _body(jj, carry):
            j = jj * 2
            pltpu.async_copy(table.at[rowb.at[j + 1]], gbufb, gsemb)
            pltpu.make_async_copy(
                table.at[rowb.at[0]], gbufa, gsema).wait()
            pltpu.sync_copy(gbufa, agg_sh.at[colb.at[j]], add=True)

            @pl.when(jj < _NB // 2 - 1)
            def _():
                pltpu.async_copy(table.at[rowb.at[j + 2]], gbufa, gsema)

            pltpu.make_async_copy(
                table.at[rowb.at[0]], gbufb, gsemb).wait()
            pltpu.sync_copy(gbufb, agg_sh.at[colb.at[j + 1]], add=True)
            return carry
        lax.fori_loop(0, _NB // 2, _body, 0)

    @pl.when(cid == 0)
    def _():
        _run(hs_ref.at[0])

    @pl.when(cid == 1)
    def _():
        _run(hs_ref.at[1])

    plsc.subcore_barrier()
    rbase = sid * (_N // _NS)
    pltpu.sync_copy(agg_sh.at[pl.ds(rbase, _N // _NS)],
                    agg_ref.at[cid, pl.ds(rbase, _N // _NS)])


@functools.cache
def _agg_kernel():
    return pl.kernel(
        _agg_body,
        out_type=jax.ShapeDtypeStruct((_NC, _N, _HD), jnp.float32),
        mesh=_sc_mesh(),
        compiler_params=pltpu.CompilerParams(use_tc_tiling_on_sc=False),
        scratch_types=[
            pltpu.VMEM((_NB, _EB), jnp.int32),
            pltpu.VMEM((_NB, _EB), jnp.int32),
            pltpu.VMEM((_EB, _HD), jnp.float32),
            pltpu.VMEM((_EB, _HD), jnp.float32),
            pltpu.VMEM_SHARED((_ROWS_PAD, _HD), jnp.float32),
            pltpu.SemaphoreType.DMA,
            pltpu.SemaphoreType.DMA,
            pltpu.SemaphoreType.DMA,
        ],
    )


def _first_body(x_ref, w_ref, b_ref, deg_ref, h0_ref, hs_ref, dinv_ref):
    h = lax.dot_general(x_ref[...], w_ref[...], (((1,), (1,)), ((), ())),
                        preferred_element_type=jnp.float32)
    h = jnp.maximum(h + b_ref[...], 0.0)
    dinv = lax.rsqrt(deg_ref[0] + deg_ref[1] + 1.0)
    h0_ref[...] = h
    hs = h * dinv
    hs_ref[0] = hs[:, :_HD]
    hs_ref[1] = hs[:, _HD:]
    dinv_ref[...] = dinv


@functools.cache
def _first_kernel():
    return pl.pallas_call(
        _first_body,
        grid=(_N // _RB,),
        in_specs=[
            pl.BlockSpec((_RB, _D), lambda i: (i, 0)),
            pl.BlockSpec((_D, _D), lambda i: (0, 0)),
            pl.BlockSpec((1, _D), lambda i: (0, 0)),
            pl.BlockSpec((_NC, _RB, 1), lambda i: (0, i, 0)),
        ],
        out_specs=[
            pl.BlockSpec((_RB, _D), lambda i: (i, 0)),
            pl.BlockSpec((_NC, _RB, _HD), lambda i: (0, i, 0)),
            pl.BlockSpec((_RB, 1), lambda i: (i, 0)),
        ],
        out_shape=[
            jax.ShapeDtypeStruct((_N, _D), jnp.float32),
            jax.ShapeDtypeStruct((_NC, _N, _HD), jnp.float32),
            jax.ShapeDtypeStruct((_N, 1), jnp.float32),
        ],
    )


def _layer_body(agg_ref, hsp_ref, x0_ref, dinv_ref, w_ref, g_ref, be_ref,
                out_ref, *, bcoef, last):
    a = jnp.concatenate(
        [agg_ref[0] + hsp_ref[0], agg_ref[1] + hsp_ref[1]], axis=1)
    dinv = dinv_ref[...]
    support = (1.0 - _ALPHA) * (a * dinv) + _ALPHA * x0_ref[...]
    out = (1.0 - bcoef) * support + bcoef * lax.dot_general(
        support, w_ref[...], (((1,), (0,)), ((), ())),
        preferred_element_type=jnp.float32)
    mu = jnp.mean(out, axis=-1, keepdims=True)
    d = out - mu
    var = jnp.mean(d * d, axis=-1, keepdims=True)
    out = d * lax.rsqrt(var + 1e-5) * g_ref[...] + be_ref[...]
    if last:
        out_ref[...] = out
    else:
        out = jnp.maximum(out, 0.0)
        hs = out * dinv
        out_ref[0] = hs[:, :_HD]
        out_ref[1] = hs[:, _HD:]


@functools.cache
def _layer_kernel(l):
    bcoef = float(np.log(_THETA / (l + 1) + 1.0))
    last = l == _L - 1
    if last:
        out_spec = pl.BlockSpec((_RB, _D), lambda i: (i, 0))
        out_shape = jax.ShapeDtypeStruct((_N, _D), jnp.float32)
    else:
        out_spec = pl.BlockSpec((_NC, _RB, _HD), lambda i: (0, i, 0))
        out_shape = jax.ShapeDtypeStruct((_NC, _N, _HD), jnp.float32)
    return pl.pallas_call(
        functools.partial(_layer_body, bcoef=bcoef, last=last),
        grid=(_N // _RB,),
        in_specs=[
            pl.BlockSpec((_NC, _RB, _HD), lambda i: (0, i, 0)),
            pl.BlockSpec((_NC, _RB, _HD), lambda i: (0, i, 0)),
            pl.BlockSpec((_RB, _D), lambda i: (i, 0)),
            pl.BlockSpec((_RB, 1), lambda i: (i, 0)),
            pl.BlockSpec((_D, _D), lambda i: (0, 0)),
            pl.BlockSpec((1, _D), lambda i: (0, 0)),
            pl.BlockSpec((1, _D), lambda i: (0, 0)),
        ],
        out_specs=out_spec,
        out_shape=out_shape,
    )


def kernel(x, edge_index, W_in, b_in, conv_W, ln_gamma, ln_beta):
    per_w = _E // _NS
    rowp = jnp.pad(edge_index[0].reshape(_NS, per_w),
                   ((0, 0), (0, _ETILE - per_w))
                   ).reshape(_NS, _NB, _EB)
    colp = jnp.pad(edge_index[1].reshape(_NS, per_w),
                   ((0, 0), (0, _ETILE - per_w)),
                   constant_values=_GARBAGE
                   ).reshape(_NS, _NB, _EB)

    colp2 = jnp.pad(colp.reshape(_NS, _ETILE),
                    ((0, 0), (0, _DTILE - _ETILE)),
                    constant_values=_GARBAGE).reshape(_NS, _NC, _NSBC, _SBE)
    deg = _deg_kernel()(colp2)[:, :_N, 0:1]
    h0, hs, dinv = _first_kernel()(x, W_in, b_in.reshape(1, _D), deg)

    out = None
    for l in range(_L):
        agg = _agg_kernel()(hs, rowp, colp)
        step = _layer_kernel(l)(
            agg, hs, h0, dinv, conv_W[l],
            ln_gamma[l].reshape(1, _D), ln_beta[l].reshape(1, _D))
        if l == _L - 1:
            out = step
        else:
            hs = step
    return out
